# baseline (speedup 1.0000x reference)
import functools

import jax
import jax.numpy as jnp
from jax import lax
from jax.experimental import pallas as pl
from jax.experimental.pallas import tpu as pltpu
from jax.experimental.pallas import tpu_sc as plsc

_NUM_CORES = 2
_NUM_SUBCORES = 16
_NUM_WORKERS = _NUM_CORES * _NUM_SUBCORES

_VOCAB = 1_000_000
_D = 64
_W = 256
_NU = _VOCAB // _W
_TAIL_V0 = _NU * _W
_TAIL_W = _VOCAB - _TAIL_V0

_CHUNK = 256
_MESH = plsc.VectorSubcoreMesh(core_axis_name="c", subcore_axis_name="s")
_PARAMS = pltpu.CompilerParams(
    use_tc_tiling_on_sc=True, needs_layout_passes=False
)


def _worker_id():
    return lax.axis_index("s") * _NUM_CORES + lax.axis_index("c")


@functools.partial(
    pl.kernel,
    mesh=_MESH,
    out_type=jax.ShapeDtypeStruct((_VOCAB // 2, 2 * _D), jnp.float32),
    scratch_types=[
        pltpu.VMEM((2, _D, _W), jnp.float32),
        pltpu.VMEM((2, _W // 2, 2 * _D), jnp.float32),
        pltpu.VMEM((_D, _TAIL_W), jnp.float32),
        pltpu.SemaphoreType.DMA,
        pltpu.SemaphoreType.DMA,
        pltpu.SemaphoreType.DMA,
        pltpu.SemaphoreType.DMA,
    ],
    compiler_params=_PARAMS,
)
def _transpose_table(tt_hbm, tp_hbm, inb, outb, tailb, is0, is1, os0, os1):
    wid = _worker_id()
    isems = (is0, is1)
    osems = (os0, os1)
    lanes = lax.iota(jnp.int32, 16)

    n_my = (_NU - wid + _NUM_WORKERS - 1) // _NUM_WORKERS

    def unit_of(t):
        return wid + t * _NUM_WORKERS

    def fire_in(t, slot):
        pltpu.async_copy(
            tt_hbm.at[:, pl.ds(unit_of(t) * _W, _W)], inb.at[slot], isems[slot]
        )

    def wait_in(slot):
        pltpu.make_async_copy(
            tt_hbm.at[:, pl.ds(0, _W)], inb.at[slot], isems[slot]
        ).wait()

    def fire_out(t, slot):
        pltpu.async_copy(
            outb.at[slot],
            tp_hbm.at[pl.ds(unit_of(t) * (_W // 2), _W // 2)],
            osems[slot],
        )

    def wait_out(slot):
        pltpu.make_async_copy(
            outb.at[slot], tp_hbm.at[pl.ds(0, _W // 2)], osems[slot]
        ).wait()

    def transpose_from(src, half_rows):
        @plsc.parallel_loop(0, 2 * half_rows // 16, unroll=4)
        def grp(c0):
            vvs = lanes + c0 * 16
            row_idx = lax.shift_right_logical(vvs, 1)
            col0 = lax.mul(lax.bitwise_and(vvs, 1), _D)
            for f in range(_D):
                vals = src[0][f, pl.ds(c0 * 16, 16)]
                plsc.store_scatter(src[1], [row_idx, col0 + f], vals)

    def transpose_unit(slot, half_rows):
        transpose_from((inb.at[slot], outb.at[slot]), half_rows)

    fire_in(0, 0)
    fire_in(1, 1)

    def step(g, _):
        for slot in (0, 1):
            t = 2 * g + slot

            @pl.when(t < n_my)
            def _():
                @pl.when(t >= 2)
                def _():
                    wait_out(slot)

                wait_in(slot)
                transpose_unit(slot, _W // 2)

                @pl.when(t + 2 < n_my)
                def _():
                    fire_in(t + 2, slot)

                fire_out(t, slot)

        return 0

    lax.fori_loop(0, (n_my + 1) // 2, step, 0)
    wait_out(0)
    wait_out(1)

    @pl.when(wid == 0)
    def _():
        pltpu.sync_copy(tt_hbm.at[:, pl.ds(_TAIL_V0, _TAIL_W)], tailb)
        transpose_from((tailb, outb.at[0]), _TAIL_W // 2)
        pltpu.sync_copy(
            outb.at[0, pl.ds(0, _TAIL_W // 2)],
            tp_hbm.at[pl.ds(_TAIL_V0 // 2, _TAIL_W // 2)],
        )


def _make_gather(n_seq, n_tok):
    n_chunks_per_seq = n_tok // _CHUNK
    n_tasks = n_seq * n_chunks_per_seq
    n_my = n_tasks // _NUM_WORKERS

    @functools.partial(
        pl.kernel,
        mesh=_MESH,
        out_type=jax.ShapeDtypeStruct((n_seq, _D, n_tok), jnp.float32),
        scratch_types=[
            pltpu.VMEM((2, _CHUNK), jnp.int32),
            pltpu.VMEM((2, _CHUNK // 128, 128), jnp.int32),
            pltpu.VMEM((2, _CHUNK, 2 * _D), jnp.float32),
            pltpu.VMEM((2, _D, _CHUNK), jnp.float32),
            pltpu.SemaphoreType.DMA,
            pltpu.SemaphoreType.DMA,
            pltpu.SemaphoreType.DMA,
            pltpu.SemaphoreType.DMA,
            pltpu.SemaphoreType.DMA,
            pltpu.SemaphoreType.DMA,
        ],
        compiler_params=_PARAMS,
    )
    def _gather(
        tp_hbm, xt_hbm, out_hbm, idxb, pairb, rows, outb,
        ix0, ix1, g0, g1, w0, w1,
    ):
        wid = _worker_id()
        ixsems = (ix0, ix1)
        gsems = (g0, g1)
        wsems = (w0, w1)
        lanes = lax.iota(jnp.int32, 16)

        def task_of(t):
            return wid * n_my + t

        def ji(t):
            c = task_of(t)
            return c // n_chunks_per_seq, (c % n_chunks_per_seq) * _CHUNK

        def fire_idx(t, slot):
            j, i0 = ji(t)
            pltpu.async_copy(
                xt_hbm.at[j, pl.ds(i0, _CHUNK)], idxb.at[slot], ixsems[slot]
            )

        def wait_idx(slot):
            pltpu.make_async_copy(
                xt_hbm.at[0, pl.ds(0, _CHUNK)], idxb.at[slot], ixsems[slot]
            ).wait()

        def fire_gather(slot):
            @plsc.parallel_loop(0, _CHUNK // 16, unroll=4)
            def grp(k):
                v = idxb[slot, pl.ds(k * 16, 16)]
                h = k // 8
                pairb[slot, h, pl.ds((k % 8) * 16, 16)] = (
                    lax.shift_right_logical(v, 1)
                )
            for h in range(_CHUNK // 128):
                pltpu.async_copy(
                    tp_hbm.at[pairb.at[slot, h]],
                    rows.at[slot, pl.ds(h * 128, 128)],
                    gsems[slot],
                )

        def wait_gather(slot):
            for h in range(_CHUNK // 128):
                pltpu.make_async_copy(
                    tp_hbm.at[pairb.at[slot, h]],
                    rows.at[slot, pl.ds(h * 128, 128)],
                    gsems[slot],
                ).wait()

        def fire_out(t, slot):
            j, i0 = ji(t)
            pltpu.async_copy(
                outb.at[slot],
                out_hbm.at[j, :, pl.ds(i0, _CHUNK)],
                wsems[slot],
            )

        def wait_out(slot):
            pltpu.make_async_copy(
                outb.at[slot], out_hbm.at[0, :, pl.ds(0, _CHUNK)], wsems[slot]
            ).wait()

        def select_transpose(slot):
            @plsc.parallel_loop(0, _CHUNK // 16, unroll=4)
            def grp(k):
                v = idxb[slot, pl.ds(k * 16, 16)]
                colbase = lax.mul(lax.bitwise_and(v, 1), _D)
                tok = lanes + k * 16
                for f in range(_D):
                    g = plsc.load_gather(
                        rows.at[slot], [tok, colbase + f]
                    )
                    outb[slot, f, pl.ds(k * 16, 16)] = g

        fire_idx(0, 0)
        if n_my > 1:
            fire_idx(1, 1)
        wait_idx(0)
        fire_gather(0)

        def step(g, _):
            for slot in (0, 1):
                t = 2 * g + slot
                nxt = 1 - slot

                @pl.when(t + 1 < n_my)
                def _():
                    wait_idx(nxt)
                    fire_gather(nxt)

                wait_gather(slot)

                @pl.when(t >= 2)
                def _():
                    wait_out(slot)

                select_transpose(slot)
                fire_out(t, slot)

                @pl.when(t + 2 < n_my)
                def _():
                    fire_idx(t + 2, slot)

            return 0

        lax.fori_loop(0, n_my // 2, step, 0)
        wait_out(0)
        wait_out(1)

    return _gather


@jax.jit
def _embed(x, table):
    n_tok, n_seq = x.shape
    tt = table.T
    xt = x.T.astype(jnp.int32)
    tp = _transpose_table(tt)
    p = _make_gather(n_seq, n_tok)(tp, xt)
    return p.transpose(2, 0, 1)


def kernel(x, table):
    return _embed(x, table)


# speedup vs baseline: 1.8062x; 1.8062x over previous
import functools

import jax
import jax.numpy as jnp
from jax import lax
from jax.experimental import pallas as pl
from jax.experimental.pallas import tpu as pltpu
from jax.experimental.pallas import tpu_sc as plsc

_NUM_CORES = 2
_NUM_SUBCORES = 16
_NUM_WORKERS = _NUM_CORES * _NUM_SUBCORES

_VOCAB = 1_000_000
_D = 64
_CHUNK = 128
_NSUB = _CHUNK // 128
_MESH = plsc.VectorSubcoreMesh(core_axis_name="c", subcore_axis_name="s")
_PARAMS = pltpu.CompilerParams(
    use_tc_tiling_on_sc=True, needs_layout_passes=False
)


def _make_gather(n_seq, n_tok):
    n_chunks_per_seq = n_tok // _CHUNK
    n_tasks = n_seq * n_chunks_per_seq
    n_my = n_tasks // _NUM_WORKERS

    @functools.partial(
        pl.kernel,
        mesh=_MESH,
        out_type=jax.ShapeDtypeStruct((n_seq, n_tok, _D), jnp.float32),
        scratch_types=[
            pltpu.VMEM((2, _NSUB, 128), jnp.int32),
            pltpu.VMEM((2, _CHUNK, 2 * _D), jnp.float32),
            pltpu.VMEM((2, _CHUNK, _D), jnp.float32),
            pltpu.SemaphoreType.DMA,
            pltpu.SemaphoreType.DMA,
            pltpu.SemaphoreType.DMA,
            pltpu.SemaphoreType.DMA,
            pltpu.SemaphoreType.DMA,
            pltpu.SemaphoreType.DMA,
        ],
        compiler_params=_PARAMS,
    )
    def _gather(
        td_hbm, xt_hbm, out_hbm, idxb, rows, outb,
        ix0, ix1, g0, g1, w0, w1,
    ):
        wid = _worker_id()
        ixsems = (ix0, ix1)
        gsems = (g0, g1)
        wsems = (w0, w1)

        def ji(t):
            c = wid * n_my + t
            return c // n_chunks_per_seq, (c % n_chunks_per_seq) * _CHUNK

        def fire_idx(t, slot):
            j, i0 = ji(t)
            for h in range(_NSUB):
                pltpu.async_copy(
                    xt_hbm.at[j, pl.ds(i0 + h * 128, 128)],
                    idxb.at[slot, h],
                    ixsems[slot],
                )
        def wait_idx(slot):
            for h in range(_NSUB):
                pltpu.make_async_copy(
                    xt_hbm.at[0, pl.ds(0, 128)],
                    idxb.at[slot, h],
                    ixsems[slot],
                ).wait()

        def fire_gather(slot):
            for h in range(_NSUB):
                pltpu.async_copy(
                    td_hbm.at[idxb.at[slot, h]],
                    rows.at[slot, pl.ds(h * 128, 128)],
                    gsems[slot],
                )

        def wait_gather(slot):
            for h in range(_NSUB):
                pltpu.make_async_copy(
                    td_hbm.at[idxb.at[slot, h]],
                    rows.at[slot, pl.ds(h * 128, 128)],
                    gsems[slot],
                ).wait()

        def fire_out(t, slot):
            j, i0 = ji(t)
            pltpu.async_copy(
                outb.at[slot],
                out_hbm.at[j, pl.ds(i0, _CHUNK)],
                wsems[slot],
            )

        def wait_out(slot):
            pltpu.make_async_copy(
                outb.at[slot], out_hbm.at[0, pl.ds(0, _CHUNK)], wsems[slot]
            ).wait()

        def select(slot):
            @plsc.parallel_loop(0, _CHUNK // 16, unroll=2)
            def grp(g):
                h = g // (128 // 16)
                v16 = idxb[slot, h, pl.ds(lax.rem(g, 128 // 16) * 16, 16)]
                s16 = lax.mul(lax.bitwise_and(v16, 1), _D)
                for cc in range(16):
                    c = g * 16 + cc
                    s = s16[cc]
                    for k in range(_D // 16):
                        outb[slot, c, pl.ds(k * 16, 16)] = rows[
                            slot, c, pl.ds(s + k * 16, 16)
                        ]

        fire_idx(0, 0)
        fire_idx(1, 1)
        wait_idx(0)
        fire_gather(0)

        def step(g, _):
            for slot in (0, 1):
                t = 2 * g + slot
                nxt = 1 - slot

                @pl.when(t + 1 < n_my)
                def _():
                    wait_idx(nxt)
                    fire_gather(nxt)

                wait_gather(slot)

                @pl.when(t >= 2)
                def _():
                    wait_out(slot)

                select(slot)
                fire_out(t, slot)

                @pl.when(t + 2 < n_my)
                def _():
                    fire_idx(t + 2, slot)

            return 0

        lax.fori_loop(0, n_my // 2, step, 0)
        wait_out(0)
        wait_out(1)

    return _gather


def _worker_id():
    return lax.axis_index("s") * _NUM_CORES + lax.axis_index("c")


@jax.jit
def _embed(x, table):
    n_tok, n_seq = x.shape
    tdup = jnp.concatenate([table, table], axis=1)
    xt = x.T.astype(jnp.int32)
    p = _make_gather(n_seq, n_tok)(tdup, xt)
    return p.transpose(1, 0, 2)


def kernel(x, table):
    return _embed(x, table)
